# Initial kernel scaffold; baseline (speedup 1.0000x reference)
import jax
import jax.numpy as jnp
from jax.experimental import pallas as pl


def kernel(xyz, feature, raw_neighbors_feature, neighbors_idx, W_nb, b_nb, g_nb, be_nb, W_attn, W_out, b_out, g_out, be_out, W_sc, b_sc, g_sc, be_sc):
    raise NotImplementedError("write your pallas kernel here")


# speedup vs baseline: 7.9640x; 7.9640x over previous
import functools

import jax
import jax.numpy as jnp
from jax import lax
from jax.experimental import pallas as pl
from jax.experimental.pallas import tpu as pltpu
from jax.experimental.pallas import tpu_sc as plsc

_NC, _NS = 2, 16
_NW = _NC * _NS
_EPS = 1e-5
_SLOPE = 0.2


def _leaky(x):
    return jnp.where(x >= 0, x, _SLOPE * x)


def _sc_gather(table, flat_idx):
    T = flat_idx.shape[0]
    D = table.shape[1]
    per_w = T // _NW
    GROUP = 1000
    n_groups = per_w // GROUP
    assert per_w % GROUP == 0 and GROUP % 8 == 0
    subs = []
    off = 0
    while off < GROUP:
        ln = min(128, GROUP - off)
        subs.append((off, ln))
        off += ln

    mesh = plsc.VectorSubcoreMesh(core_axis_name="c", subcore_axis_name="s")

    @functools.partial(
        pl.kernel,
        out_type=jax.ShapeDtypeStruct((T, D), jnp.float32),
        mesh=mesh,
        scratch_types=[
            pltpu.VMEM((GROUP,), jnp.int32),
            pltpu.VMEM((GROUP, D), jnp.float32),
            pltpu.SemaphoreType.DMA,
        ],
        compiler_params=pltpu.CompilerParams(use_tc_tiling_on_sc=False),
    )
    def k(table_hbm, idx_hbm, out_hbm, idx_v, rows_v, sem):
        wid = lax.axis_index("s") * _NC + lax.axis_index("c")
        base = wid * per_w

        @pl.loop(0, n_groups)
        def _(g):
            off = base + g * GROUP
            pltpu.sync_copy(idx_hbm.at[pl.ds(off, GROUP)], idx_v)
            cps = [
                pltpu.async_copy(
                    table_hbm.at[idx_v.at[pl.ds(o, ln)]],
                    rows_v.at[pl.ds(o, ln)],
                    sem,
                )
                for o, ln in subs
            ]
            for cp in cps:
                cp.wait()
            pltpu.sync_copy(rows_v, out_hbm.at[pl.ds(off, GROUP)])

    return k(table, flat_idx)


def _p1_body(f_ref, r_ref, wsc_ref, bsc_ref, wnb_ref, bnb_ref,
             short_ref, shs_ref, nbs_ref):
    sl = jnp.dot(f_ref[...], wsc_ref[...],
                 preferred_element_type=jnp.float32) + bsc_ref[...]
    short_ref[...] = sl
    nl = jnp.dot(r_ref[...], wnb_ref[...],
                 preferred_element_type=jnp.float32) + bnb_ref[...]

    @pl.when(pl.program_id(0) == 0)
    def _():
        shs_ref[...] = jnp.zeros_like(shs_ref)
        nbs_ref[...] = jnp.zeros_like(nbs_ref)

    shs_ref[...] += jnp.stack([jnp.sum(sl, 0), jnp.sum(sl * sl, 0)])
    nbs_ref[...] += jnp.stack([jnp.sum(nl, 0), jnp.sum(nl * nl, 0)])


def _pass1(feat2d, raw2d, W_sc, b_sc, W_nb, b_nb, interpret=False):
    M, Cin = feat2d.shape
    Mk, Cr = raw2d.shape
    K = Mk // M
    Csc = W_sc.shape[1]
    Cnb = W_nb.shape[1]
    RB = 2000
    G = M // RB
    return pl.pallas_call(
        _p1_body,
        grid=(G,),
        in_specs=[
            pl.BlockSpec((RB, Cin), lambda i: (i, 0)),
            pl.BlockSpec((RB * K, Cr), lambda i: (i, 0)),
            pl.BlockSpec((Cin, Csc), lambda i: (0, 0)),
            pl.BlockSpec((1, Csc), lambda i: (0, 0)),
            pl.BlockSpec((Cr, Cnb), lambda i: (0, 0)),
            pl.BlockSpec((1, Cnb), lambda i: (0, 0)),
        ],
        out_specs=[
            pl.BlockSpec((RB, Csc), lambda i: (i, 0)),
            pl.BlockSpec((2, Csc), lambda i: (0, 0)),
            pl.BlockSpec((2, Cnb), lambda i: (0, 0)),
        ],
        out_shape=[
            jax.ShapeDtypeStruct((M, Csc), jnp.float32),
            jax.ShapeDtypeStruct((2, Csc), jnp.float32),
            jax.ShapeDtypeStruct((2, Cnb), jnp.float32),
        ],
        interpret=interpret,
    )(feat2d, raw2d, W_sc, b_sc.reshape(1, -1), W_nb, b_nb.reshape(1, -1))


def _p2_body(K, RB, g_ref, r_ref, wnb_ref, cnb_ref, wat_ref, wout_ref,
             bout_ref, main_ref, ms_ref):
    nl = jnp.dot(r_ref[...], wnb_ref[...],
                 preferred_element_type=jnp.float32) + cnb_ref[...]
    nb = _leaky(nl)
    feat = jnp.concatenate([g_ref[...], nb], axis=1)
    logits = jnp.dot(feat, wat_ref[...], preferred_element_type=jnp.float32)
    C = logits.shape[1]
    l3 = logits.reshape(RB, K, C)
    m = jnp.max(l3, axis=1, keepdims=True)
    e = jnp.exp(l3 - m)
    a = e / jnp.sum(e, axis=1, keepdims=True)
    pooled = jnp.sum(a * feat.reshape(RB, K, C), axis=1)
    ml = jnp.dot(pooled, wout_ref[...],
                 preferred_element_type=jnp.float32) + bout_ref[...]
    main_ref[...] = ml

    @pl.when(pl.program_id(0) == 0)
    def _():
        ms_ref[...] = jnp.zeros_like(ms_ref)

    ms_ref[...] += jnp.stack([jnp.sum(ml, 0), jnp.sum(ml * ml, 0)])


def _pass2(gath2d, raw2d, Wnb_f, cnb, W_attn, W_out, b_out, M, interpret=False):
    Mk, Cin = gath2d.shape
    Cr = raw2d.shape[1]
    K = Mk // M
    Cnb = Wnb_f.shape[1]
    Cat = W_attn.shape[1]
    Co = W_out.shape[1]
    RB = 1000
    G = M // RB
    return pl.pallas_call(
        functools.partial(_p2_body, K, RB),
        grid=(G,),
        in_specs=[
            pl.BlockSpec((RB * K, Cin), lambda i: (i, 0)),
            pl.BlockSpec((RB * K, Cr), lambda i: (i, 0)),
            pl.BlockSpec((Cr, Cnb), lambda i: (0, 0)),
            pl.BlockSpec((1, Cnb), lambda i: (0, 0)),
            pl.BlockSpec((Cin + Cnb, Cat), lambda i: (0, 0)),
            pl.BlockSpec((Cat, Co), lambda i: (0, 0)),
            pl.BlockSpec((1, Co), lambda i: (0, 0)),
        ],
        out_specs=[
            pl.BlockSpec((RB, Co), lambda i: (i, 0)),
            pl.BlockSpec((2, Co), lambda i: (0, 0)),
        ],
        out_shape=[
            jax.ShapeDtypeStruct((M, Co), jnp.float32),
            jax.ShapeDtypeStruct((2, Co), jnp.float32),
        ],
        interpret=interpret,
    )(gath2d, raw2d, Wnb_f, cnb.reshape(1, -1), W_attn, W_out,
      b_out.reshape(1, -1))


def _p3_body(s_ref, m_ref, asc_ref, csc_ref, ao_ref, co_ref, o_ref):
    o_ref[...] = _leaky(s_ref[...] * asc_ref[...] + csc_ref[...]
                        + m_ref[...] * ao_ref[...] + co_ref[...])


def _pass3(short_lin, main_lin, a_sc, c_sc, a_o, c_o, interpret=False):
    M, C = short_lin.shape
    RB = 5000
    G = M // RB
    return pl.pallas_call(
        _p3_body,
        grid=(G,),
        in_specs=[
            pl.BlockSpec((RB, C), lambda i: (i, 0)),
            pl.BlockSpec((RB, C), lambda i: (i, 0)),
            pl.BlockSpec((1, C), lambda i: (0, 0)),
            pl.BlockSpec((1, C), lambda i: (0, 0)),
            pl.BlockSpec((1, C), lambda i: (0, 0)),
            pl.BlockSpec((1, C), lambda i: (0, 0)),
        ],
        out_specs=pl.BlockSpec((RB, C), lambda i: (i, 0)),
        out_shape=jax.ShapeDtypeStruct((M, C), jnp.float32),
        interpret=interpret,
    )(short_lin, main_lin, a_sc.reshape(1, -1), c_sc.reshape(1, -1),
      a_o.reshape(1, -1), c_o.reshape(1, -1))


def kernel(xyz, feature, raw_neighbors_feature, neighbors_idx,
           W_nb, b_nb, g_nb, be_nb, W_attn,
           W_out, b_out, g_out, be_out,
           W_sc, b_sc, g_sc, be_sc):
    B, N, Cin = feature.shape
    K = neighbors_idx.shape[2]
    Cr = raw_neighbors_feature.shape[3]
    M = B * N
    Mk = M * K

    feat2d = feature.reshape(M, Cin)
    raw2d = raw_neighbors_feature.reshape(Mk, Cr)
    flat_idx = (neighbors_idx.astype(jnp.int32)
                + (jnp.arange(B, dtype=jnp.int32) * N)[:, None, None]
                ).reshape(Mk)

    gathered = _sc_gather(feat2d, flat_idx)
    short_lin, sh_stats, nb_stats = _pass1(feat2d, raw2d, W_sc, b_sc,
                                           W_nb, b_nb)

    nb_mean = nb_stats[0] / Mk
    nb_var = nb_stats[1] / Mk - nb_mean * nb_mean
    s_nb = g_nb * lax.rsqrt(nb_var + _EPS)
    Wnb_f = W_nb * s_nb[None, :]
    cnb = (b_nb - nb_mean) * s_nb + be_nb

    main_lin, m_stats = _pass2(gathered, raw2d, Wnb_f, cnb,
                               W_attn, W_out, b_out, M)

    sh_mean = sh_stats[0] / M
    sh_var = sh_stats[1] / M - sh_mean * sh_mean
    a_sc = g_sc * lax.rsqrt(sh_var + _EPS)
    c_sc = be_sc - sh_mean * a_sc
    m_mean = m_stats[0] / M
    m_var = m_stats[1] / M - m_mean * m_mean
    a_o = g_out * lax.rsqrt(m_var + _EPS)
    c_o = be_out - m_mean * a_o

    out2d = _pass3(short_lin, main_lin, a_sc, c_sc, a_o, c_o)
    out = out2d.reshape(B, N, -1)
    return (xyz, out, raw_neighbors_feature, neighbors_idx)


# speedup vs baseline: 10.1710x; 1.2771x over previous
import functools

import jax
import jax.numpy as jnp
from jax import lax
from jax.experimental import pallas as pl
from jax.experimental.pallas import tpu as pltpu
from jax.experimental.pallas import tpu_sc as plsc

_NC, _NS = 2, 16
_NW = _NC * _NS
_EPS = 1e-5
_SLOPE = 0.2


def _leaky(x):
    return jnp.where(x >= 0, x, _SLOPE * x)


def _sc_gather(table, flat_idx):
    T = flat_idx.shape[0]
    D = table.shape[1]
    per_w = T // _NW
    GROUP = 1000
    n_groups = per_w // GROUP
    assert per_w % GROUP == 0 and GROUP % 8 == 0
    subs = []
    off = 0
    while off < GROUP:
        ln = min(128, GROUP - off)
        subs.append((off, ln))
        off += ln

    mesh = plsc.VectorSubcoreMesh(core_axis_name="c", subcore_axis_name="s")

    @functools.partial(
        pl.kernel,
        out_type=jax.ShapeDtypeStruct((T, 128), jnp.float32),
        mesh=mesh,
        scratch_types=[
            pltpu.VMEM((GROUP,), jnp.int32),
            pltpu.VMEM((GROUP, D), jnp.float32),
            pltpu.SemaphoreType.DMA,
        ],
        compiler_params=pltpu.CompilerParams(use_tc_tiling_on_sc=False),
    )
    def k(table_hbm, idx_hbm, out_hbm, idx_v, rows_v, sem):
        wid = lax.axis_index("s") * _NC + lax.axis_index("c")
        base = wid * per_w

        @pl.loop(0, n_groups)
        def _(g):
            off = base + g * GROUP
            pltpu.sync_copy(idx_hbm.at[pl.ds(off, GROUP)], idx_v)
            cps = [
                pltpu.async_copy(
                    table_hbm.at[idx_v.at[pl.ds(o, ln)]],
                    rows_v.at[pl.ds(o, ln)],
                    sem,
                )
                for o, ln in subs
            ]
            for cp in cps:
                cp.wait()
            pltpu.sync_copy(rows_v,
                            out_hbm.at[pl.ds(off, GROUP), pl.ds(0, D)])

    return k(table, flat_idx)


def _p1_body(f_ref, r_ref, wsc_ref, bsc_ref, wnb_ref, bnb_ref,
             short_ref, shs_ref, nbs_ref):
    sl = jnp.dot(f_ref[...], wsc_ref[...],
                 preferred_element_type=jnp.float32) + bsc_ref[...]
    short_ref[...] = sl
    nl = jnp.dot(r_ref[...], wnb_ref[...],
                 preferred_element_type=jnp.float32) + bnb_ref[...]

    @pl.when(pl.program_id(0) == 0)
    def _():
        shs_ref[...] = jnp.zeros_like(shs_ref)
        nbs_ref[...] = jnp.zeros_like(nbs_ref)

    shs_ref[...] += jnp.stack([jnp.sum(sl, 0), jnp.sum(sl * sl, 0)])
    nbs_ref[...] += jnp.stack([jnp.sum(nl, 0), jnp.sum(nl * nl, 0)])


def _pass1(feat2d, raw2d, W_sc, b_sc, W_nb, b_nb, interpret=False):
    M, Cin = feat2d.shape
    Mk, Cr = raw2d.shape
    K = Mk // M
    Csc = W_sc.shape[1]
    Cnb = W_nb.shape[1]
    RB = 2000
    G = M // RB
    return pl.pallas_call(
        _p1_body,
        grid=(G,),
        in_specs=[
            pl.BlockSpec((RB, Cin), lambda i: (i, 0)),
            pl.BlockSpec((RB * K, Cr), lambda i: (i, 0)),
            pl.BlockSpec((Cin, Csc), lambda i: (0, 0)),
            pl.BlockSpec((1, Csc), lambda i: (0, 0)),
            pl.BlockSpec((Cr, Cnb), lambda i: (0, 0)),
            pl.BlockSpec((1, Cnb), lambda i: (0, 0)),
        ],
        out_specs=[
            pl.BlockSpec((RB, Csc), lambda i: (i, 0)),
            pl.BlockSpec((2, Csc), lambda i: (0, 0)),
            pl.BlockSpec((2, Cnb), lambda i: (0, 0)),
        ],
        out_shape=[
            jax.ShapeDtypeStruct((M, Csc), jnp.float32),
            jax.ShapeDtypeStruct((2, Csc), jnp.float32),
            jax.ShapeDtypeStruct((2, Cnb), jnp.float32),
        ],
        interpret=interpret,
    )(feat2d, raw2d, W_sc, b_sc.reshape(1, -1), W_nb, b_nb.reshape(1, -1))


def _p2_body(K, RB, Cin, g_ref, r_ref, wnb_ref, cnb_ref, wat_ref, wout_ref,
             bout_ref, main_ref, ms_ref):
    nl = jnp.dot(r_ref[...], wnb_ref[...],
                 preferred_element_type=jnp.float32) + cnb_ref[...]
    nb = _leaky(nl)
    feat = jnp.concatenate([g_ref[:, :Cin], nb], axis=1)
    logits = jnp.dot(feat, wat_ref[...], preferred_element_type=jnp.float32)
    C = logits.shape[1]
    l3 = logits.reshape(RB, K, C)
    m = jnp.max(l3, axis=1, keepdims=True)
    e = jnp.exp(l3 - m)
    a = e / jnp.sum(e, axis=1, keepdims=True)
    pooled = jnp.sum(a * feat.reshape(RB, K, C), axis=1)
    ml = jnp.dot(pooled, wout_ref[...],
                 preferred_element_type=jnp.float32) + bout_ref[...]
    main_ref[...] = ml

    @pl.when(pl.program_id(0) == 0)
    def _():
        ms_ref[...] = jnp.zeros_like(ms_ref)

    ms_ref[...] += jnp.stack([jnp.sum(ml, 0), jnp.sum(ml * ml, 0)])


def _pass2(gath2d, raw2d, Wnb_f, cnb, W_attn, W_out, b_out, M, Cin,
           interpret=False):
    Mk, Cg = gath2d.shape
    Cr = raw2d.shape[1]
    K = Mk // M
    Cnb = Wnb_f.shape[1]
    Cat = W_attn.shape[1]
    Co = W_out.shape[1]
    RB = 1000
    G = M // RB
    return pl.pallas_call(
        functools.partial(_p2_body, K, RB, Cin),
        grid=(G,),
        in_specs=[
            pl.BlockSpec((RB * K, Cg), lambda i: (i, 0)),
            pl.BlockSpec((RB * K, Cr), lambda i: (i, 0)),
            pl.BlockSpec((Cr, Cnb), lambda i: (0, 0)),
            pl.BlockSpec((1, Cnb), lambda i: (0, 0)),
            pl.BlockSpec((Cin + Cnb, Cat), lambda i: (0, 0)),
            pl.BlockSpec((Cat, Co), lambda i: (0, 0)),
            pl.BlockSpec((1, Co), lambda i: (0, 0)),
        ],
        out_specs=[
            pl.BlockSpec((RB, Co), lambda i: (i, 0)),
            pl.BlockSpec((2, Co), lambda i: (0, 0)),
        ],
        out_shape=[
            jax.ShapeDtypeStruct((M, Co), jnp.float32),
            jax.ShapeDtypeStruct((2, Co), jnp.float32),
        ],
        interpret=interpret,
    )(gath2d, raw2d, Wnb_f, cnb.reshape(1, -1), W_attn, W_out,
      b_out.reshape(1, -1))


def _p3_body(s_ref, m_ref, asc_ref, csc_ref, ao_ref, co_ref, o_ref):
    o_ref[...] = _leaky(s_ref[...] * asc_ref[...] + csc_ref[...]
                        + m_ref[...] * ao_ref[...] + co_ref[...])


def _pass3(short_lin, main_lin, a_sc, c_sc, a_o, c_o, interpret=False):
    M, C = short_lin.shape
    RB = 5000
    G = M // RB
    return pl.pallas_call(
        _p3_body,
        grid=(G,),
        in_specs=[
            pl.BlockSpec((RB, C), lambda i: (i, 0)),
            pl.BlockSpec((RB, C), lambda i: (i, 0)),
            pl.BlockSpec((1, C), lambda i: (0, 0)),
            pl.BlockSpec((1, C), lambda i: (0, 0)),
            pl.BlockSpec((1, C), lambda i: (0, 0)),
            pl.BlockSpec((1, C), lambda i: (0, 0)),
        ],
        out_specs=pl.BlockSpec((RB, C), lambda i: (i, 0)),
        out_shape=jax.ShapeDtypeStruct((M, C), jnp.float32),
        interpret=interpret,
    )(short_lin, main_lin, a_sc.reshape(1, -1), c_sc.reshape(1, -1),
      a_o.reshape(1, -1), c_o.reshape(1, -1))


def kernel(xyz, feature, raw_neighbors_feature, neighbors_idx,
           W_nb, b_nb, g_nb, be_nb, W_attn,
           W_out, b_out, g_out, be_out,
           W_sc, b_sc, g_sc, be_sc):
    B, N, Cin = feature.shape
    K = neighbors_idx.shape[2]
    Cr = raw_neighbors_feature.shape[3]
    M = B * N
    Mk = M * K

    feat2d = feature.reshape(M, Cin)
    raw2d = raw_neighbors_feature.reshape(Mk, Cr)
    flat_idx = (neighbors_idx.astype(jnp.int32)
                + (jnp.arange(B, dtype=jnp.int32) * N)[:, None, None]
                ).reshape(Mk)

    gathered = _sc_gather(feat2d, flat_idx)
    short_lin, sh_stats, nb_stats = _pass1(feat2d, raw2d, W_sc, b_sc,
                                           W_nb, b_nb)

    nb_mean = nb_stats[0] / Mk
    nb_var = nb_stats[1] / Mk - nb_mean * nb_mean
    s_nb = g_nb * lax.rsqrt(nb_var + _EPS)
    Wnb_f = W_nb * s_nb[None, :]
    cnb = (b_nb - nb_mean) * s_nb + be_nb

    main_lin, m_stats = _pass2(gathered, raw2d, Wnb_f, cnb,
                               W_attn, W_out, b_out, M, Cin)

    sh_mean = sh_stats[0] / M
    sh_var = sh_stats[1] / M - sh_mean * sh_mean
    a_sc = g_sc * lax.rsqrt(sh_var + _EPS)
    c_sc = be_sc - sh_mean * a_sc
    m_mean = m_stats[0] / M
    m_var = m_stats[1] / M - m_mean * m_mean
    a_o = g_out * lax.rsqrt(m_var + _EPS)
    c_o = be_out - m_mean * a_o

    out2d = _pass3(short_lin, main_lin, a_sc, c_sc, a_o, c_o)
    out = out2d.reshape(B, N, -1)
    return (xyz, out, raw_neighbors_feature, neighbors_idx)
